# baseline (speedup 1.0000x reference)
import functools

import jax
import jax.numpy as jnp
from jax import lax
from jax.experimental import pallas as pl
from jax.experimental.pallas import tpu as pltpu
from jax.experimental.pallas import tpu_sc as plsc

N = 10000
E = 320000
IN_CH = 128
HID = 256
OUT_CH = 16
NG = 64

NC, NS = 2, 16
NW = NC * NS
CB = 80
EC = E // CB
NPAD = 10240
RPT = NPAD // NS
BT = 2000
GRID = N // BT
CPTA = EC // NS
CPTH = EC // NW
NB = 4
IC = 25


def _sc_agg_body(OUTER):
    def body(table, gidx, cidx, zeros, out, shared, gidx_v, cidx_v, rows_v,
             gsem, ssem):
        c = lax.axis_index("c")
        s = lax.axis_index("s")
        wid = c * NS + s
        pltpu.sync_copy(zeros, shared.at[pl.ds(s * RPT, RPT)])
        plsc.subcore_barrier()

        def buf(q):
            return rows_v.at[pl.ds(lax.rem(q, NB) * CB, CB)]

        def fire_gather(q):
            pltpu.async_copy(table.at[gidx_v.at[q]], buf(q), gsem)

        def fire_scatter(q):
            pltpu.async_copy(buf(q), shared.at[cidx_v.at[q]], ssem, add=True)

        def absorb(sem):
            pltpu.make_async_copy(table.at[pl.ds(0, CB)],
                                  rows_v.at[pl.ds(0, CB)], sem).wait()

        def outer(o, carry):
            blk = wid * OUTER + o
            pltpu.sync_copy(gidx.at[blk], gidx_v)
            pltpu.sync_copy(cidx.at[blk], cidx_v)
            fire_gather(0)
            fire_gather(1)

            def inner(j, carry2):
                @pl.when(j >= 2)
                def _():
                    absorb(ssem)

                @pl.when(j <= IC - 1)
                def _():
                    absorb(gsem)
                    fire_scatter(j)

                @pl.when(j <= IC - 3)
                def _():
                    fire_gather(j + 2)

                return carry2

            lax.fori_loop(0, IC + 2, inner, 0)
            return carry

        lax.fori_loop(0, OUTER, outer, 0)
        plsc.subcore_barrier()
        pltpu.sync_copy(shared.at[pl.ds(s * RPT, RPT)], out.at[wid])

    return body


@functools.lru_cache(maxsize=None)
def _make_sc_agg(OUTER):
    return functools.partial(
        pl.kernel,
        out_type=jax.ShapeDtypeStruct((NW, RPT, 128), jnp.float32),
        mesh=plsc.VectorSubcoreMesh(core_axis_name="c", subcore_axis_name="s",
                                    num_cores=NC, num_subcores=NS),
        scratch_types=[
            pltpu.VMEM_SHARED((NPAD, 128), jnp.float32),
            pltpu.VMEM((IC, CB), jnp.int32),
            pltpu.VMEM((IC, CB), jnp.int32),
            pltpu.VMEM((NB * CB, 128), jnp.float32),
            pltpu.SemaphoreType.DMA,
            pltpu.SemaphoreType.DMA,
        ],
    )(_sc_agg_body(OUTER))


def _sc_hist_body(cidx, ones, zeros, out, shared, cidx_v, ones_v, ssem):
    OUTER = CPTH // IC
    c = lax.axis_index("c")
    s = lax.axis_index("s")
    wid = c * NS + s
    pltpu.sync_copy(zeros, shared.at[pl.ds(s * RPT, RPT)])
    pltpu.sync_copy(ones, ones_v)
    plsc.subcore_barrier()

    def absorb():
        pltpu.make_async_copy(zeros.at[pl.ds(0, CB)], ones_v, ssem).wait()

    def outer(o, carry):
        pltpu.sync_copy(cidx.at[wid * OUTER + o], cidx_v)

        def step(j, carry2):
            pltpu.async_copy(ones_v, shared.at[cidx_v.at[j]], ssem, add=True)

            @pl.when(j >= 4)
            def _():
                absorb()

            return carry2

        lax.fori_loop(0, IC, step, 0)
        for _ in range(4):
            absorb()
        return carry

    lax.fori_loop(0, OUTER, outer, 0)
    plsc.subcore_barrier()
    pltpu.sync_copy(shared.at[pl.ds(s * RPT, RPT)], out.at[wid])


@functools.lru_cache(maxsize=None)
def _make_sc_hist():
    return functools.partial(
        pl.kernel,
        out_type=jax.ShapeDtypeStruct((NW, RPT, 128), jnp.float32),
        mesh=plsc.VectorSubcoreMesh(core_axis_name="c", subcore_axis_name="s",
                                    num_cores=NC, num_subcores=NS),
        scratch_types=[
            pltpu.VMEM_SHARED((NPAD, 128), jnp.float32),
            pltpu.VMEM((IC, CB), jnp.int32),
            pltpu.VMEM((CB, 128), jnp.float32),
            pltpu.SemaphoreType.DMA,
        ],
    )(_sc_hist_body)


def _h3(c):
    return pl.BlockSpec((1, BT, 128), lambda i, c=c: (c, i, 0))


def _dinv_of(h0_ref, h1_ref):
    deg = h0_ref[0, :, 0:1] + h1_ref[0, :, 0:1] + 1.0
    return lax.rsqrt(deg)


def _tc1_body(x_ref, h0_ref, h1_ref, o_ref):
    o_ref[...] = x_ref[...] * _dinv_of(h0_ref, h1_ref)


def _tc1(x, hist3):
    bs = lambda shp: pl.BlockSpec(shp, lambda i: (i, 0))
    return pl.pallas_call(
        _tc1_body,
        grid=(GRID,),
        in_specs=[bs((BT, IN_CH)), _h3(0), _h3(1)],
        out_specs=bs((BT, IN_CH)),
        out_shape=jax.ShapeDtypeStruct((N, IN_CH), jnp.float32),
    )(x, hist3, hist3)


def _tc2_body(p0_ref, p1_ref, xs_ref, h0_ref, h1_ref,
              w1_ref, b1_ref, olo_ref, ohi_ref):
    dinv = _dinv_of(h0_ref, h1_ref)
    y1 = (p0_ref[0] + p1_ref[0] + xs_ref[...]) * dinv
    h = jax.nn.relu(
        jnp.dot(y1, w1_ref[...], preferred_element_type=jnp.float32)
        + b1_ref[...])
    xs2 = h * dinv
    olo_ref[...] = xs2[:, :128]
    ohi_ref[...] = xs2[:, 128:]


def _tc2(agg13, xs1, hist3, W1, b1):
    bs = lambda shp: pl.BlockSpec(shp, lambda i: (i, 0))
    full = lambda shp: pl.BlockSpec(shp, lambda i: (0, 0))
    return pl.pallas_call(
        _tc2_body,
        grid=(GRID,),
        in_specs=[_h3(0), _h3(1), bs((BT, 128)), _h3(0), _h3(1),
                  full((IN_CH, HID)), full((1, HID))],
        out_specs=[bs((BT, 128)), bs((BT, 128))],
        out_shape=[jax.ShapeDtypeStruct((N, 128), jnp.float32)] * 2,
    )(agg13, agg13, xs1, hist3, hist3, W1, b1)


def _tc3_body(alo_ref, ahi_ref, xlo_ref, xhi_ref, h0_ref, h1_ref, b_ref,
              w2_ref, b2_ref, wl_ref, bl_ref, out_ref, acc_ref, cnt_ref):
    i = pl.program_id(0)
    dinv = _dinv_of(h0_ref, h1_ref)
    y2 = jnp.concatenate(
        [alo_ref[0] + xlo_ref[...], ahi_ref[0] + xhi_ref[...]], axis=1)
    y2 = y2 * dinv
    h = jax.nn.relu(
        jnp.dot(y2, w2_ref[...], preferred_element_type=jnp.float32)
        + b2_ref[...])
    seg = b_ref[0, 0, :]
    onehot = (seg[None, :] ==
              lax.broadcasted_iota(jnp.int32, (NG, BT), 0)).astype(jnp.float32)
    psum = jnp.dot(onehot, h, preferred_element_type=jnp.float32)
    pcnt = jnp.sum(onehot, axis=1, keepdims=True)

    @pl.when(i == 0)
    def _():
        acc_ref[...] = psum
        cnt_ref[...] = pcnt

    @pl.when(i > 0)
    def _():
        acc_ref[...] += psum
        cnt_ref[...] += pcnt

    @pl.when(i == GRID - 1)
    def _():
        pooled = acc_ref[...] / jnp.maximum(cnt_ref[...], 1.0)
        out_ref[...] = (
            jnp.dot(pooled, wl_ref[...], preferred_element_type=jnp.float32)
            + bl_ref[...])


def _tc3(agg23, xs2lo, xs2hi, hist3, batch3, W2, b2, Wl, bl):
    bs = lambda shp: pl.BlockSpec(shp, lambda i: (i, 0))
    full = lambda shp: pl.BlockSpec(shp, lambda i: (0, 0))
    return pl.pallas_call(
        _tc3_body,
        grid=(GRID,),
        in_specs=[_h3(0), _h3(1), bs((BT, 128)), bs((BT, 128)),
                  _h3(0), _h3(1),
                  pl.BlockSpec((1, 1, BT), lambda i: (i, 0, 0)),
                  full((HID, HID)), full((1, HID)),
                  full((HID, OUT_CH)), full((1, OUT_CH))],
        out_specs=pl.BlockSpec((NG, OUT_CH), lambda i: (0, 0)),
        out_shape=jax.ShapeDtypeStruct((NG, OUT_CH), jnp.float32),
        scratch_shapes=[pltpu.VMEM((NG, HID), jnp.float32),
                        pltpu.VMEM((NG, 1), jnp.float32)],
    )(agg23, agg23, xs2lo, xs2hi, hist3, hist3, batch3, W2, b2, Wl, bl)


def kernel(x, edge_index, batch, W1, b1, W2, b2, Wl, bl):
    ei = edge_index.astype(jnp.int32)
    rowh = ei[0].reshape(NW * (CPTH // IC), IC, CB)
    colh = ei[1].reshape(NW * (CPTH // IC), IC, CB)
    row3 = ei[0].reshape(NS, CPTA, CB)
    col3 = ei[1].reshape(NS, CPTA, CB)
    gidx2 = jnp.concatenate([row3, row3 + N], axis=0).reshape(
        NW * (CPTA // IC), IC, CB)
    cidx2 = jnp.concatenate([col3, col3], axis=0).reshape(
        NW * (CPTA // IC), IC, CB)
    batch3 = batch.astype(jnp.int32).reshape(GRID, 1, BT)
    zeros128 = jnp.zeros((RPT, 128), jnp.float32)
    ones128 = jnp.ones((CB, 128), jnp.float32)

    hist3 = _make_sc_hist()(colh, ones128, zeros128).reshape(NC, NPAD, 128)
    xs1 = _tc1(x, hist3)
    agg13 = _make_sc_agg(CPTH // IC)(xs1, rowh, colh, zeros128).reshape(
        NC, NPAD, 128)
    xs2lo, xs2hi = _tc2(agg13, xs1, hist3, W1, b1.reshape(1, HID))
    xs2cat = jnp.concatenate([xs2lo, xs2hi], axis=0)
    agg23 = _make_sc_agg(CPTA // IC)(xs2cat, gidx2, cidx2, zeros128).reshape(
        NC, NPAD, 128)
    return _tc3(agg23, xs2lo, xs2hi, hist3, batch3, W2,
                b2.reshape(1, HID), Wl, bl.reshape(1, OUT_CH))


# speedup vs baseline: 1.0588x; 1.0588x over previous
import functools

import jax
import jax.numpy as jnp
from jax import lax
from jax.experimental import pallas as pl
from jax.experimental.pallas import tpu as pltpu
from jax.experimental.pallas import tpu_sc as plsc

N = 10000
E = 320000
IN_CH = 128
HID = 256
OUT_CH = 16
NG = 64

NC, NS = 2, 16
NW = NC * NS
CB = 80
EC = E // CB
NPAD = 10240
RPT = NPAD // NS
BT = 2000
GRID = N // BT
CPTA = EC // NS
CPTH = EC // NW
NB = 4
IC = 25


def _sc_agg_body(OUTER):
    def body(table, gidx, cidx, zeros, out, shared, gidx_v, cidx_v, rows_v,
             gsem, ssem):
        c = lax.axis_index("c")
        s = lax.axis_index("s")
        wid = c * NS + s
        pltpu.sync_copy(zeros, shared.at[pl.ds(s * RPT, RPT)])
        plsc.subcore_barrier()

        def buf(q):
            return rows_v.at[pl.ds(lax.rem(q, NB) * CB, CB)]

        def fire_gather(q):
            pltpu.async_copy(table.at[gidx_v.at[q]], buf(q), gsem)

        def fire_scatter(q):
            pltpu.async_copy(buf(q), shared.at[cidx_v.at[q]], ssem, add=True)

        def absorb(sem):
            pltpu.make_async_copy(table.at[pl.ds(0, CB)],
                                  rows_v.at[pl.ds(0, CB)], sem).wait()

        def outer(o, carry):
            blk = wid * OUTER + o
            pltpu.sync_copy(gidx.at[blk], gidx_v)
            pltpu.sync_copy(cidx.at[blk], cidx_v)
            fire_gather(0)
            fire_gather(1)

            def inner(j, carry2):
                @pl.when(j >= 2)
                def _():
                    absorb(ssem)

                @pl.when(j <= IC - 3)
                def _():
                    fire_gather(j + 2)

                @pl.when(j <= IC - 1)
                def _():
                    absorb(gsem)
                    fire_scatter(j)

                return carry2

            lax.fori_loop(0, IC + 2, inner, 0)
            return carry

        lax.fori_loop(0, OUTER, outer, 0)
        plsc.subcore_barrier()
        pltpu.sync_copy(shared.at[pl.ds(s * RPT, RPT)], out.at[wid])

    return body


@functools.lru_cache(maxsize=None)
def _make_sc_agg(OUTER):
    return functools.partial(
        pl.kernel,
        out_type=jax.ShapeDtypeStruct((NW, RPT, 128), jnp.float32),
        mesh=plsc.VectorSubcoreMesh(core_axis_name="c", subcore_axis_name="s",
                                    num_cores=NC, num_subcores=NS),
        scratch_types=[
            pltpu.VMEM_SHARED((NPAD, 128), jnp.float32),
            pltpu.VMEM((IC, CB), jnp.int32),
            pltpu.VMEM((IC, CB), jnp.int32),
            pltpu.VMEM((NB * CB, 128), jnp.float32),
            pltpu.SemaphoreType.DMA,
            pltpu.SemaphoreType.DMA,
        ],
    )(_sc_agg_body(OUTER))


def _sc_hist_body(cidx, ones, zeros, out, shared, cidx_v, ones_v, ssem):
    OUTER = CPTH // IC
    c = lax.axis_index("c")
    s = lax.axis_index("s")
    wid = c * NS + s
    pltpu.sync_copy(zeros, shared.at[pl.ds(s * RPT, RPT)])
    pltpu.sync_copy(ones, ones_v)
    plsc.subcore_barrier()

    def absorb():
        pltpu.make_async_copy(zeros.at[pl.ds(0, CB)], ones_v, ssem).wait()

    def outer(o, carry):
        pltpu.sync_copy(cidx.at[wid * OUTER + o], cidx_v)

        def step(j, carry2):
            pltpu.async_copy(ones_v, shared.at[cidx_v.at[j]], ssem, add=True)

            @pl.when(j >= 4)
            def _():
                absorb()

            return carry2

        lax.fori_loop(0, IC, step, 0)
        for _ in range(4):
            absorb()
        return carry

    lax.fori_loop(0, OUTER, outer, 0)
    plsc.subcore_barrier()
    pltpu.sync_copy(shared.at[pl.ds(s * RPT, RPT)], out.at[wid])


@functools.lru_cache(maxsize=None)
def _make_sc_hist():
    return functools.partial(
        pl.kernel,
        out_type=jax.ShapeDtypeStruct((NW, RPT, 128), jnp.float32),
        mesh=plsc.VectorSubcoreMesh(core_axis_name="c", subcore_axis_name="s",
                                    num_cores=NC, num_subcores=NS),
        scratch_types=[
            pltpu.VMEM_SHARED((NPAD, 128), jnp.float32),
            pltpu.VMEM((IC, CB), jnp.int32),
            pltpu.VMEM((CB, 128), jnp.float32),
            pltpu.SemaphoreType.DMA,
        ],
    )(_sc_hist_body)


def _h3(c):
    return pl.BlockSpec((1, BT, 128), lambda i, c=c: (c, i, 0))


def _dinv_of(h0_ref, h1_ref):
    deg = h0_ref[0, :, 0:1] + h1_ref[0, :, 0:1] + 1.0
    return lax.rsqrt(deg)


def _tc1_body(x_ref, h0_ref, h1_ref, o_ref):
    o_ref[...] = x_ref[...] * _dinv_of(h0_ref, h1_ref)


def _tc1(x, hist3):
    bs = lambda shp: pl.BlockSpec(shp, lambda i: (i, 0))
    return pl.pallas_call(
        _tc1_body,
        grid=(GRID,),
        in_specs=[bs((BT, IN_CH)), _h3(0), _h3(1)],
        out_specs=bs((BT, IN_CH)),
        out_shape=jax.ShapeDtypeStruct((N, IN_CH), jnp.float32),
    )(x, hist3, hist3)


def _tc2_body(p0_ref, p1_ref, xs_ref, h0_ref, h1_ref,
              w1_ref, b1_ref, olo_ref, ohi_ref):
    dinv = _dinv_of(h0_ref, h1_ref)
    y1 = (p0_ref[0] + p1_ref[0] + xs_ref[...]) * dinv
    h = jax.nn.relu(
        jnp.dot(y1, w1_ref[...], preferred_element_type=jnp.float32)
        + b1_ref[...])
    xs2 = h * dinv
    olo_ref[...] = xs2[:, :128]
    ohi_ref[...] = xs2[:, 128:]


def _tc2(agg13, xs1, hist3, W1, b1):
    bs = lambda shp: pl.BlockSpec(shp, lambda i: (i, 0))
    full = lambda shp: pl.BlockSpec(shp, lambda i: (0, 0))
    return pl.pallas_call(
        _tc2_body,
        grid=(GRID,),
        in_specs=[_h3(0), _h3(1), bs((BT, 128)), _h3(0), _h3(1),
                  full((IN_CH, HID)), full((1, HID))],
        out_specs=[bs((BT, 128)), bs((BT, 128))],
        out_shape=[jax.ShapeDtypeStruct((N, 128), jnp.float32)] * 2,
    )(agg13, agg13, xs1, hist3, hist3, W1, b1)


def _tc3_body(alo_ref, ahi_ref, xlo_ref, xhi_ref, h0_ref, h1_ref, b_ref,
              w2_ref, b2_ref, wl_ref, bl_ref, out_ref, acc_ref, cnt_ref):
    i = pl.program_id(0)
    dinv = _dinv_of(h0_ref, h1_ref)
    y2 = jnp.concatenate(
        [alo_ref[0] + xlo_ref[...], ahi_ref[0] + xhi_ref[...]], axis=1)
    y2 = y2 * dinv
    h = jax.nn.relu(
        jnp.dot(y2, w2_ref[...], preferred_element_type=jnp.float32)
        + b2_ref[...])
    seg = b_ref[0, 0, :]
    onehot = (seg[None, :] ==
              lax.broadcasted_iota(jnp.int32, (NG, BT), 0)).astype(jnp.float32)
    psum = jnp.dot(onehot, h, preferred_element_type=jnp.float32)
    pcnt = jnp.sum(onehot, axis=1, keepdims=True)

    @pl.when(i == 0)
    def _():
        acc_ref[...] = psum
        cnt_ref[...] = pcnt

    @pl.when(i > 0)
    def _():
        acc_ref[...] += psum
        cnt_ref[...] += pcnt

    @pl.when(i == GRID - 1)
    def _():
        pooled = acc_ref[...] / jnp.maximum(cnt_ref[...], 1.0)
        out_ref[...] = (
            jnp.dot(pooled, wl_ref[...], preferred_element_type=jnp.float32)
            + bl_ref[...])


def _tc3(agg23, xs2lo, xs2hi, hist3, batch3, W2, b2, Wl, bl):
    bs = lambda shp: pl.BlockSpec(shp, lambda i: (i, 0))
    full = lambda shp: pl.BlockSpec(shp, lambda i: (0, 0))
    return pl.pallas_call(
        _tc3_body,
        grid=(GRID,),
        in_specs=[_h3(0), _h3(1), bs((BT, 128)), bs((BT, 128)),
                  _h3(0), _h3(1),
                  pl.BlockSpec((1, 1, BT), lambda i: (i, 0, 0)),
                  full((HID, HID)), full((1, HID)),
                  full((HID, OUT_CH)), full((1, OUT_CH))],
        out_specs=pl.BlockSpec((NG, OUT_CH), lambda i: (0, 0)),
        out_shape=jax.ShapeDtypeStruct((NG, OUT_CH), jnp.float32),
        scratch_shapes=[pltpu.VMEM((NG, HID), jnp.float32),
                        pltpu.VMEM((NG, 1), jnp.float32)],
    )(agg23, agg23, xs2lo, xs2hi, hist3, hist3, batch3, W2, b2, Wl, bl)


def kernel(x, edge_index, batch, W1, b1, W2, b2, Wl, bl):
    ei = edge_index.astype(jnp.int32)
    rowh = ei[0].reshape(NW * (CPTH // IC), IC, CB)
    colh = ei[1].reshape(NW * (CPTH // IC), IC, CB)
    row3 = ei[0].reshape(NS, CPTA, CB)
    col3 = ei[1].reshape(NS, CPTA, CB)
    gidx2 = jnp.concatenate([row3, row3 + N], axis=0).reshape(
        NW * (CPTA // IC), IC, CB)
    cidx2 = jnp.concatenate([col3, col3], axis=0).reshape(
        NW * (CPTA // IC), IC, CB)
    batch3 = batch.astype(jnp.int32).reshape(GRID, 1, BT)
    zeros128 = jnp.zeros((RPT, 128), jnp.float32)
    ones128 = jnp.ones((CB, 128), jnp.float32)

    hist3 = _make_sc_hist()(colh, ones128, zeros128).reshape(NC, NPAD, 128)
    xs1 = _tc1(x, hist3)
    agg13 = _make_sc_agg(CPTH // IC)(xs1, rowh, colh, zeros128).reshape(
        NC, NPAD, 128)
    xs2lo, xs2hi = _tc2(agg13, xs1, hist3, W1, b1.reshape(1, HID))
    xs2cat = jnp.concatenate([xs2lo, xs2hi], axis=0)
    agg23 = _make_sc_agg(CPTA // IC)(xs2cat, gidx2, cidx2, zeros128).reshape(
        NC, NPAD, 128)
    return _tc3(agg23, xs2lo, xs2hi, hist3, batch3, W2,
                b2.reshape(1, HID), Wl, bl.reshape(1, OUT_CH))
